# baseline (speedup 1.0000x reference)
import functools

import jax
import jax.numpy as jnp
from jax.experimental import pallas as pl
from jax.experimental.pallas import tpu as pltpu


def _gcn_fused_kernel(x_ref, w1_ref, b1_ref, w2_ref, b2_ref, o_ref,
                      y_scr, *, ks, H, W):
    pad = ks // 2
    HW = H * W
    CHUNKS = 4
    MB = HW // CHUNKS
    HB = H // CHUNKS
    x = x_ref[0]

    y_scr[:pad] = jnp.zeros_like(y_scr[:pad])
    y_scr[W + pad:] = jnp.zeros_like(y_scr[W + pad:])

    WB = W // CHUNKS
    for s in range(x_ref.shape[0]):
        x = x_ref[s]

        for m in range(CHUNKS):
            r0 = m * MB
            xs = jnp.concatenate(
                [x[k * W + r0:k * W + r0 + MB] for k in range(ks)], axis=1)
            y = jnp.dot(xs, w1_ref[...],
                        preferred_element_type=jnp.float32) + b1_ref[...]
            yt = jnp.transpose(
                y.astype(jnp.bfloat16).reshape(HB, W, -1), (1, 0, 2))
            y_scr[pad:W + pad, m * HB:(m + 1) * HB] = yt

        for m in range(CHUNKS):
            w0 = m * WB
            ys = jnp.concatenate(
                [y_scr[j + w0:j + w0 + WB].reshape(MB, -1)
                 for j in range(ks)], axis=1)
            o_ref[s, m * MB:(m + 1) * MB] = (
                jnp.dot(ys, w2_ref[...],
                        preferred_element_type=jnp.float32)
                + b2_ref[...]).astype(jnp.bfloat16)


def kernel(x, conv1_1_w, conv1_1_b, conv1_2_w, conv1_2_b,
           conv2_1_w, conv2_1_b, conv2_2_w, conv2_2_b):
    N, C_in, H, W = x.shape
    C_out = conv1_1_w.shape[0]
    ks = conv1_1_w.shape[2]
    pad = ks // 2
    HW = H * W

    w1 = jnp.concatenate([conv1_1_w[:, :, :, 0], conv2_1_w[:, :, :, 0]],
                         axis=0)
    w1 = jnp.transpose(w1, (2, 1, 0)).astype(jnp.bfloat16)
    w1 = w1.reshape(ks * C_in, 2 * C_out)
    b1 = jnp.concatenate([conv1_1_b, conv2_1_b])[None, :]

    w2 = jnp.concatenate([conv1_2_w[:, :, 0, :], conv2_2_w[:, :, 0, :]],
                         axis=1)
    w2 = jnp.transpose(w2, (2, 1, 0)).astype(jnp.bfloat16)
    w2 = w2.reshape(ks * 2 * C_out, C_out)
    b2 = (conv1_2_b + conv2_2_b)[None, :]

    xt = jnp.transpose(x.astype(jnp.bfloat16), (0, 2, 3, 1))
    xt = jnp.pad(xt, ((0, 0), (pad, pad), (0, 0), (0, 0)))
    xt = xt.reshape(N, (H + 2 * pad) * W, C_in)

    _kfn = functools.partial(_gcn_fused_kernel, ks=ks, H=H, W=W)
    out = pl.pallas_call(
        _kfn,
        out_shape=jax.ShapeDtypeStruct((N, HW, C_out), jnp.bfloat16),
        grid=(N // 2,),
        in_specs=[
            pl.BlockSpec((2, (H + 2 * pad) * W, C_in), lambda n: (n, 0, 0)),
            pl.BlockSpec((ks * C_in, 2 * C_out), lambda n: (0, 0)),
            pl.BlockSpec((1, 2 * C_out), lambda n: (0, 0)),
            pl.BlockSpec((ks * 2 * C_out, C_out), lambda n: (0, 0)),
            pl.BlockSpec((1, C_out), lambda n: (0, 0)),
        ],
        out_specs=pl.BlockSpec((2, HW, C_out), lambda n: (n, 0, 0)),
        scratch_shapes=[pltpu.VMEM((W + 2 * pad, H, 2 * C_out),
                                   jnp.bfloat16)],
        compiler_params=pltpu.CompilerParams(
            dimension_semantics=("parallel",),
            allow_input_fusion=(True, False, False, False, False)),
    )(xt, w1, b1, w2, b2)

    return jnp.transpose(out.reshape(N, W, H, C_out),
                         (0, 3, 2, 1)).astype(jnp.float32)


# speedup vs baseline: 1.0999x; 1.0999x over previous
import functools

import jax
import jax.numpy as jnp
from jax.experimental import pallas as pl
from jax.experimental.pallas import tpu as pltpu


def _gcn_fused_kernel(x_ref, w1_ref, b1_ref, w2_ref, b2_ref, o_ref,
                      y_scr, *, ks, H, W):
    pad = ks // 2
    HW = H * W
    CHUNKS = 4
    MB = HW // CHUNKS
    HB = H // CHUNKS
    x = x_ref[0]

    y_scr[:pad] = jnp.zeros_like(y_scr[:pad])
    y_scr[W + pad:] = jnp.zeros_like(y_scr[W + pad:])

    WB = W // CHUNKS
    for s in range(x_ref.shape[0]):
        x = x_ref[s]

        for m in range(CHUNKS):
            r0 = m * MB
            xs = jnp.concatenate(
                [x[k * W + r0:k * W + r0 + MB] for k in range(ks)], axis=1)
            y = jnp.dot(xs, w1_ref[...],
                        preferred_element_type=jnp.float32) + b1_ref[...]
            yt = jnp.transpose(
                y.astype(jnp.bfloat16).reshape(HB, W, -1), (1, 0, 2))
            y_scr[pad:W + pad, m * HB:(m + 1) * HB] = yt

        for m in range(CHUNKS):
            w0 = m * WB
            ys = jnp.concatenate(
                [y_scr[j + w0:j + w0 + WB].reshape(MB, -1)
                 for j in range(ks)], axis=1)
            o_ref[s, m * MB:(m + 1) * MB] = (
                jnp.dot(ys, w2_ref[...],
                        preferred_element_type=jnp.float32) + b2_ref[...])


def kernel(x, conv1_1_w, conv1_1_b, conv1_2_w, conv1_2_b,
           conv2_1_w, conv2_1_b, conv2_2_w, conv2_2_b):
    N, C_in, H, W = x.shape
    C_out = conv1_1_w.shape[0]
    ks = conv1_1_w.shape[2]
    pad = ks // 2
    HW = H * W

    w1 = jnp.concatenate([conv1_1_w[:, :, :, 0], conv2_1_w[:, :, :, 0]],
                         axis=0)
    w1 = jnp.transpose(w1, (2, 1, 0)).astype(jnp.bfloat16)
    w1 = w1.reshape(ks * C_in, 2 * C_out)
    b1 = jnp.concatenate([conv1_1_b, conv2_1_b])[None, :]

    w2 = jnp.concatenate([conv1_2_w[:, :, 0, :], conv2_2_w[:, :, 0, :]],
                         axis=1)
    w2 = jnp.transpose(w2, (2, 1, 0)).astype(jnp.bfloat16)
    w2 = w2.reshape(ks * 2 * C_out, C_out)
    b2 = (conv1_2_b + conv2_2_b)[None, :]

    xt = jnp.transpose(x.astype(jnp.bfloat16), (0, 2, 3, 1))
    xt = jnp.pad(xt, ((0, 0), (pad, pad), (0, 0), (0, 0)))
    xt = xt.reshape(N, (H + 2 * pad) * W, C_in)

    _kfn = functools.partial(_gcn_fused_kernel, ks=ks, H=H, W=W)
    out = pl.pallas_call(
        _kfn,
        out_shape=jax.ShapeDtypeStruct((N, HW, C_out), jnp.float32),
        grid=(N // 2,),
        in_specs=[
            pl.BlockSpec((2, (H + 2 * pad) * W, C_in), lambda n: (n, 0, 0)),
            pl.BlockSpec((ks * C_in, 2 * C_out), lambda n: (0, 0)),
            pl.BlockSpec((1, 2 * C_out), lambda n: (0, 0)),
            pl.BlockSpec((ks * 2 * C_out, C_out), lambda n: (0, 0)),
            pl.BlockSpec((1, C_out), lambda n: (0, 0)),
        ],
        out_specs=pl.BlockSpec((2, HW, C_out), lambda n: (n, 0, 0)),
        scratch_shapes=[pltpu.VMEM((W + 2 * pad, H, 2 * C_out),
                                   jnp.bfloat16)],
        compiler_params=pltpu.CompilerParams(
            dimension_semantics=("parallel",),
            allow_input_fusion=(True, False, False, False, False)),
    )(xt, w1, b1, w2, b2)

    return jnp.transpose(out.reshape(N, W, H, C_out), (0, 3, 2, 1))
